# baseline (speedup 1.0000x reference)
import functools

import jax
import jax.numpy as jnp
from jax import lax
from jax.experimental import pallas as pl
from jax.experimental.pallas import tpu as pltpu
from jax.experimental.pallas import tpu_sc as plsc

NG = 1000
LN = 50000
PN = 100000
LE = 800000
PE = 1600000

LNP = 51200
PNP = 102400
LEP = 819200
PEP = 1638400

_MESH = dict(core_axis_name="c", subcore_axis_name="s", num_cores=2,
             num_subcores=16)

_ZROWS = 64


def _make_sc_scatter(n_pad, e_pad, eb, w, cpc, split):
    nout = 2 if split else 2 * cpc
    epb = e_pad // 32 if split else e_pad // 16
    nblk = epb // eb
    rps = n_pad // 16
    nz = rps // _ZROWS

    @functools.partial(
        pl.kernel,
        out_type=jax.ShapeDtypeStruct((nout * n_pad, w), jnp.float32),
        mesh=plsc.VectorSubcoreMesh(**_MESH),
        compiler_params=pltpu.CompilerParams(use_tc_tiling_on_sc=False),
        scratch_types=[
            pltpu.VMEM((2, eb), jnp.int32),
            pltpu.VMEM((2, eb), jnp.int32),
            pltpu.VMEM((2, eb, w), jnp.float32),
            pltpu.VMEM((_ZROWS, w), jnp.float32),
            pltpu.VMEM_SHARED((n_pad, w), jnp.float32),
            pltpu.SemaphoreType.DMA,
            pltpu.SemaphoreType.DMA,
        ],
    )
    def k(table, srcs, dst, out, src_v, dst_v, rows_v, zbuf, acc, gsem, ssem):
        c = lax.axis_index("c")
        s = lax.axis_index("s")
        zero = jnp.zeros((16,), jnp.float32)

        def zb(i, carry):
            for q in range(w // 16):
                zbuf[i, pl.ds(q * 16, 16)] = zero
            return carry

        lax.fori_loop(0, _ZROWS, zb, 0)

        for cc in range(cpc):
            chunk = c * cpc + cc
            for z in range(nz):
                pltpu.sync_copy(zbuf, acc.at[pl.ds(s * rps + z * _ZROWS,
                                                   _ZROWS)])
            plsc.subcore_barrier()

            def load_and_gather(t, b):
                if split:
                    soff = c * (e_pad // 2) + s * epb + t * eb
                    doff = soff
                else:
                    soff = chunk * e_pad + s * epb + t * eb
                    doff = s * epb + t * eb
                pltpu.sync_copy(srcs.at[pl.ds(soff, eb)], src_v.at[b])
                pltpu.sync_copy(dst.at[pl.ds(doff, eb)], dst_v.at[b])
                pltpu.async_copy(table.at[src_v.at[b]], rows_v.at[b], gsem)

            def drain_scatters(b):
                pltpu.make_async_copy(rows_v.at[b], acc.at[dst_v.at[b]],
                                      ssem).wait()

            load_and_gather(0, 0)

            def eblk(t, carry):
                b = lax.rem(t, 2)
                nb = lax.rem(t + 1, 2)

                @pl.when(t >= 1)
                def _():
                    drain_scatters(nb)

                @pl.when(t + 1 < nblk)
                def _():
                    load_and_gather(t + 1, nb)

                pltpu.make_async_copy(table.at[src_v.at[b]], rows_v.at[b],
                                      gsem).wait()
                pltpu.async_copy(rows_v.at[b], acc.at[dst_v.at[b]], ssem,
                                 add=True)
                return carry

            lax.fori_loop(0, nblk, eblk, 0)
            drain_scatters((nblk - 1) % 2)
            plsc.subcore_barrier()
            oc = c if split else chunk
            pltpu.sync_copy(acc.at[pl.ds(s * rps, rps)],
                            out.at[pl.ds(oc * n_pad + s * rps, rps)])
            if cc + 1 < cpc:
                plsc.subcore_barrier()

    return k


def _make_sc_degree(db):
    rps_l = LNP // 16
    rps_p = PNP // 16

    @functools.partial(
        pl.kernel,
        out_type=(jax.ShapeDtypeStruct((2 * LNP, 16), jnp.float32),
                  jax.ShapeDtypeStruct((2 * PNP, 16), jnp.float32)),
        mesh=plsc.VectorSubcoreMesh(**_MESH),
        compiler_params=pltpu.CompilerParams(use_tc_tiling_on_sc=False),
        scratch_types=[
            pltpu.VMEM((2, db), jnp.int32),
            pltpu.VMEM((db, 16), jnp.float32),
            pltpu.VMEM((_ZROWS, 16), jnp.float32),
            pltpu.VMEM_SHARED((PNP, 16), jnp.float32),
            pltpu.SemaphoreType.DMA,
        ],
    )
    def k(dst_l, dst_p, out_l, out_p, dst_v, ones_v, zbuf, acc, ssem):
        c = lax.axis_index("c")
        s = lax.axis_index("s")
        zero = jnp.zeros((16,), jnp.float32)
        one = jnp.ones((16,), jnp.float32)

        def fill(i, carry):
            zbuf[i] = zero
            return carry

        lax.fori_loop(0, _ZROWS, fill, 0)

        def fill1(i, carry):
            ones_v[i] = one
            return carry

        lax.fori_loop(0, db, fill1, 0)

        for dst, out, rps, e_pad in ((dst_l, out_l, rps_l, LEP),
                                     (dst_p, out_p, rps_p, PEP)):
            ept = e_pad // 32
            nblk = ept // db
            nz = rps // _ZROWS
            for z in range(nz):
                pltpu.sync_copy(zbuf,
                                acc.at[pl.ds(s * rps + z * _ZROWS, _ZROWS)])
            plsc.subcore_barrier()

            def load_idx(t, b):
                doff = (c * 16 + s) * ept + t * db
                pltpu.sync_copy(dst.at[pl.ds(doff, db)], dst_v.at[b])

            def drain(b):
                pltpu.make_async_copy(ones_v, acc.at[dst_v.at[b]],
                                      ssem).wait()

            load_idx(0, 0)

            def eblk(t, carry):
                b = lax.rem(t, 2)
                nb = lax.rem(t + 1, 2)

                @pl.when(t >= 1)
                def _():
                    drain(nb)

                @pl.when(t + 1 < nblk)
                def _():
                    load_idx(t + 1, nb)

                pltpu.async_copy(ones_v, acc.at[dst_v.at[b]], ssem, add=True)
                return carry

            lax.fori_loop(0, nblk, eblk, 0)
            drain((nblk - 1) % 2)
            plsc.subcore_barrier()
            n_pad = rps * 16
            pltpu.sync_copy(acc.at[pl.ds(s * rps, rps)],
                            out.at[pl.ds(c * n_pad + s * rps, rps)])
            plsc.subcore_barrier()

    return k


def _dinv_of(deg_ref):
    d = deg_ref[0, :, 0:1] + deg_ref[1, :, 0:1] + 1.0
    return lax.rsqrt(d)


def _tc_pre1(x_ref, w_ref, deg_ref, out_ref):
    dinv = _dinv_of(deg_ref)
    h = jnp.dot(x_ref[...], w_ref[...], preferred_element_type=jnp.float32)
    g = h * dinv
    for cch in range(2):
        out_ref[cch] = g[:, cch * 32:(cch + 1) * 32]


def _tc_pre0(x_ref, deg_ref, out_ref):
    dinv = _dinv_of(deg_ref)
    g = x_ref[...] * dinv
    for cch in range(2):
        out_ref[cch] = g[:, cch * 16:(cch + 1) * 16]


def _tc_mid(acc_ref, g_ref, deg_ref, w2_ref, b1_ref, out_ref):
    dinv = _dinv_of(deg_ref)
    a = jnp.concatenate([acc_ref[i] + g_ref[i] for i in range(2)], axis=1)
    x1 = jax.nn.relu(dinv * a + b1_ref[...])
    g2 = jnp.dot(x1, w2_ref[...], preferred_element_type=jnp.float32) * dinv
    out_ref[...] = g2


def _tc_mid0(acc_ref, g_ref, deg_ref, w1_ref, w2_ref, b1_ref, out_ref):
    dinv = _dinv_of(deg_ref)
    a = jnp.concatenate([acc_ref[i] + g_ref[i] for i in range(2)], axis=1)
    z = dinv * a
    x1 = jax.nn.relu(
        jnp.dot(z, w1_ref[...], preferred_element_type=jnp.float32)
        + b1_ref[...])
    g2 = jnp.dot(x1, w2_ref[...], preferred_element_type=jnp.float32) * dinv
    for cch in range(2):
        out_ref[cch] = g2[:, cch * 16:(cch + 1) * 16]


def _make_tc_post(seg, rb, partial):
    ngb = rb // seg

    def _tc_post(acc_ref, g_ref, deg_ref, b2_ref, out_ref):
        dinv = _dinv_of(deg_ref)
        if partial:
            a = acc_ref[0] + acc_ref[1] + g_ref[...]
        else:
            a = jnp.concatenate([acc_ref[i] + g_ref[i] for i in range(2)],
                                axis=1)
        x2 = jax.nn.relu(dinv * a + b2_ref[...])
        rows = lax.broadcasted_iota(jnp.int32, (ngb, rb), 0)
        cols = lax.broadcasted_iota(jnp.int32, (ngb, rb), 1)
        pmat = jnp.where(cols // seg == rows, 1.0 / seg, 0.0)
        out_ref[...] = jnp.dot(pmat, x2, preferred_element_type=jnp.float32)

    return _tc_post


def _tc_head(xl_ref, xp_ref, w2d_ref, bilb_ref, aw1_ref, ab1_ref, aw2_ref,
             ab2_ref, ow1_ref, ob1_ref, ow2_ref, ob2_ref, fw1_ref, fb1_ref,
             fw2_ref, fb2_ref, fw3_ref, fb3_ref, out_ref):
    xl = xl_ref[...]
    xp = xp_ref[...]
    m = jnp.concatenate([xl[:, i:i + 1] * xp for i in range(32)], axis=1)
    bil = jax.nn.relu(
        jnp.dot(m, w2d_ref[...], preferred_element_type=jnp.float32)
        + bilb_ref[...])
    a1 = jax.nn.relu(
        jnp.dot(bil, aw1_ref[...], preferred_element_type=jnp.float32)
        + ab1_ref[...])
    att = jnp.sum(a1 * aw2_ref[...], axis=1, keepdims=True) + ab2_ref[...]
    attended = jax.nn.sigmoid(att) * bil
    f1 = jax.nn.relu(
        jnp.dot(attended, ow1_ref[...], preferred_element_type=jnp.float32)
        + ob1_ref[...])
    feat = jnp.dot(f1, ow2_ref[...],
                   preferred_element_type=jnp.float32) + ob2_ref[...]
    h = jax.nn.relu(
        jnp.dot(feat, fw1_ref[...], preferred_element_type=jnp.float32)
        + fb1_ref[...])
    h = jax.nn.relu(
        jnp.dot(h, fw2_ref[...], preferred_element_type=jnp.float32)
        + fb2_ref[...])
    out_ref[...] = jax.nn.sigmoid(
        jnp.sum(h * fw3_ref[...], axis=1, keepdims=True) + fb3_ref[...])


def _full(shape):
    return pl.BlockSpec(shape, lambda i: (0,) * len(shape))


def _prep_edges(ei, n, n_pad, e_pad):
    e = ei.shape[1]
    fill = jnp.full((e_pad - e,), jnp.int32(n))
    src = jnp.concatenate([ei[0], fill])
    dst2d = jnp.concatenate([ei[1], fill])

    def chunked(ch):
        return jnp.concatenate(
            [src + jnp.int32(cch * n_pad) for cch in range(ch)])

    return dst2d, chunked


def _pool(acc2, g2, deg, b2, n, n_pad, seg, partial):
    rbp = seg * 8
    deg_spec_p = pl.BlockSpec((2, rbp, 16), lambda i: (0, i, 0))
    if partial:
        acc_spec = pl.BlockSpec((2, rbp, 32), lambda i: (0, i, 0))
        g_spec = pl.BlockSpec((rbp, 32), lambda i: (i, 0))
    else:
        acc_spec = pl.BlockSpec((2, rbp, 16), lambda i: (0, i, 0))
        g_spec = acc_spec
    return pl.pallas_call(
        _make_tc_post(seg, rbp, partial),
        grid=(n // rbp,),
        in_specs=[acc_spec, g_spec, deg_spec_p, _full((1, 32))],
        out_specs=pl.BlockSpec((8, 32), lambda i: (i, 0)),
        out_shape=jax.ShapeDtypeStruct((NG, 32), jnp.float32),
    )(acc2, g2, deg, b2.reshape(1, 32))


def _ligand_branch(x, ei, W1, b1, W2, b2, deg):
    n, n_pad, e_pad, seg, rb = LN, LNP, LEP, LN // NG, 1000
    grid = n // rb
    dst2d, chunked = _prep_edges(ei, n, n_pad, e_pad)

    deg_spec = pl.BlockSpec((2, rb, 16), lambda i: (0, i, 0))
    t2w_spec = pl.BlockSpec((2, rb, 32), lambda i: (0, i, 0))
    t1w_spec = pl.BlockSpec((rb, 32), lambda i: (i, 0))

    g1 = pl.pallas_call(
        _tc_pre1,
        grid=(grid,),
        in_specs=[pl.BlockSpec((rb, 128), lambda i: (i, 0)),
                  _full((128, 64)), deg_spec],
        out_specs=t2w_spec,
        out_shape=jax.ShapeDtypeStruct((2, n_pad, 32), jnp.float32),
    )(x, W1, deg)

    acc1 = _make_sc_scatter(n_pad, e_pad, 128, 32, 1, False)(
        g1.reshape(2 * n_pad, 32), chunked(2), dst2d).reshape(2, n_pad, 32)

    g2 = pl.pallas_call(
        _tc_mid,
        grid=(grid,),
        in_specs=[t2w_spec, t2w_spec, deg_spec, _full((64, 32)),
                  _full((1, 64))],
        out_specs=t1w_spec,
        out_shape=jax.ShapeDtypeStruct((n_pad, 32), jnp.float32),
    )(acc1, g1, deg, W2, b1.reshape(1, 64))

    acc2 = _make_sc_scatter(n_pad, e_pad, 128, 32, 1, True)(
        g2, chunked(1), dst2d).reshape(2, n_pad, 32)

    return _pool(acc2, g2, deg, b2, n, n_pad, seg, True)


def _protein_branch(x, ei, W1, b1, W2, b2, deg):
    n, n_pad, e_pad, seg, rb = PN, PNP, PEP, PN // NG, 1000
    grid = n // rb
    dst2d, chunked = _prep_edges(ei, n, n_pad, e_pad)

    deg_spec = pl.BlockSpec((2, rb, 16), lambda i: (0, i, 0))
    t2_spec = pl.BlockSpec((2, rb, 16), lambda i: (0, i, 0))

    g0 = pl.pallas_call(
        _tc_pre0,
        grid=(grid,),
        in_specs=[pl.BlockSpec((rb, 32), lambda i: (i, 0)), deg_spec],
        out_specs=t2_spec,
        out_shape=jax.ShapeDtypeStruct((2, n_pad, 16), jnp.float32),
    )(x, deg)

    acc0 = _make_sc_scatter(n_pad, e_pad, 256, 16, 1, False)(
        g0.reshape(2 * n_pad, 16), chunked(2), dst2d).reshape(2, n_pad, 16)

    g2 = pl.pallas_call(
        _tc_mid0,
        grid=(grid,),
        in_specs=[t2_spec, t2_spec, deg_spec, _full((32, 64)),
                  _full((64, 32)), _full((1, 64))],
        out_specs=t2_spec,
        out_shape=jax.ShapeDtypeStruct((2, n_pad, 16), jnp.float32),
    )(acc0, g0, deg, W1, W2, b1.reshape(1, 64))

    acc2 = _make_sc_scatter(n_pad, e_pad, 256, 16, 1, False)(
        g2.reshape(2 * n_pad, 16), chunked(2), dst2d).reshape(2, n_pad, 16)

    return _pool(acc2, g2, deg, b2, n, n_pad, seg, False)


def kernel(ligand_x, protein_x, ligand_edge_index, protein_edge_index,
           ligand_batch, protein_batch, lW1, lb1, lW2, lb2, pW1, pb1, pW2,
           pb2, bilW, bilb, attW1, attb1, attW2, attb2, outW1, outb1, outW2,
           outb2, finW1, finb1, finW2, finb2, finW3, finb3):
    del ligand_batch, protein_batch

    lx = jnp.pad(ligand_x, ((0, 0), (0, 128 - 78)))
    px = jnp.pad(protein_x, ((0, 0), (0, 32 - 30)))
    lW1p = jnp.pad(lW1, ((0, 128 - 78), (0, 0)))
    pW1p = jnp.pad(pW1, ((0, 32 - 30), (0, 0)))

    ldst = jnp.concatenate(
        [ligand_edge_index[1], jnp.full((LEP - LE,), jnp.int32(LN))])
    pdst = jnp.concatenate(
        [protein_edge_index[1], jnp.full((PEP - PE,), jnp.int32(PN))])
    deg_l, deg_p = _make_sc_degree(512)(ldst, pdst)
    deg_l = deg_l.reshape(2, LNP, 16)
    deg_p = deg_p.reshape(2, PNP, 16)

    xl = _ligand_branch(lx, ligand_edge_index, lW1p, lb1, lW2, lb2, deg_l)
    xp = _protein_branch(px, protein_edge_index, pW1p, pb1, pW2, pb2, deg_p)

    w2d = bilW.transpose(1, 2, 0).reshape(32 * 32, 128)
    rb = 200
    out = pl.pallas_call(
        _tc_head,
        grid=(NG // rb,),
        in_specs=[
            pl.BlockSpec((rb, 32), lambda i: (i, 0)),
            pl.BlockSpec((rb, 32), lambda i: (i, 0)),
            _full((1024, 128)),
            _full((1, 128)),
            _full((128, 64)),
            _full((1, 64)),
            _full((1, 64)),
            _full((1, 1)),
            _full((128, 128)),
            _full((1, 128)),
            _full((128, 64)),
            _full((1, 64)),
            _full((64, 128)),
            _full((1, 128)),
            _full((128, 64)),
            _full((1, 64)),
            _full((1, 64)),
            _full((1, 1)),
        ],
        out_specs=pl.BlockSpec((rb, 1), lambda i: (i, 0)),
        out_shape=jax.ShapeDtypeStruct((NG, 1), jnp.float32),
    )(xl, xp, w2d, bilb.reshape(1, 128), attW1, attb1.reshape(1, 64),
      attW2.reshape(1, 64), attb2.reshape(1, 1), outW1, outb1.reshape(1, 128),
      outW2, outb2.reshape(1, 64), finW1, finb1.reshape(1, 128), finW2,
      finb2.reshape(1, 64), finW3.reshape(1, 64), finb3.reshape(1, 1))
    return out


# speedup vs baseline: 1.3208x; 1.3208x over previous
import functools

import jax
import jax.numpy as jnp
from jax import lax
from jax.experimental import pallas as pl
from jax.experimental.pallas import tpu as pltpu
from jax.experimental.pallas import tpu_sc as plsc

NG = 1000
LN = 50000
PN = 100000
LE = 800000
PE = 1600000

LNP = 51200
PNP = 102400
LEP = 819200
PEP = 1638400

_MESH = dict(core_axis_name="c", subcore_axis_name="s", num_cores=2,
             num_subcores=16)

_ZROWS = 64


def _make_sc_scatter(n_pad, e_pad, eb, w, cpc, split):
    nout = 2 if split else 2 * cpc
    epb = e_pad // 32 if split else e_pad // 16
    nblk = epb // eb
    rps = n_pad // 16
    nz = rps // _ZROWS

    @functools.partial(
        pl.kernel,
        out_type=jax.ShapeDtypeStruct((nout * n_pad, w), jnp.bfloat16),
        mesh=plsc.VectorSubcoreMesh(**_MESH),
        compiler_params=pltpu.CompilerParams(use_tc_tiling_on_sc=False),
        scratch_types=[
            pltpu.VMEM((2, eb), jnp.int32),
            pltpu.VMEM((2, eb), jnp.int32),
            pltpu.VMEM((2, eb, w), jnp.bfloat16),
            pltpu.VMEM((_ZROWS, w), jnp.bfloat16),
            pltpu.VMEM_SHARED((n_pad, w), jnp.bfloat16),
            pltpu.SemaphoreType.DMA,
            pltpu.SemaphoreType.DMA,
        ],
    )
    def k(table, srcs, dst, out, src_v, dst_v, rows_v, zbuf, acc, gsem, ssem):
        c = lax.axis_index("c")
        s = lax.axis_index("s")
        zero = jnp.zeros((32,), jnp.bfloat16)

        def zb(i, carry):
            for q in range(w // 32):
                zbuf[i, pl.ds(q * 32, 32)] = zero
            return carry

        lax.fori_loop(0, _ZROWS, zb, 0)

        for cc in range(cpc):
            chunk = c * cpc + cc
            for z in range(nz):
                pltpu.sync_copy(zbuf, acc.at[pl.ds(s * rps + z * _ZROWS,
                                                   _ZROWS)])
            plsc.subcore_barrier()

            def load_and_gather(t, b):
                if split:
                    soff = c * (e_pad // 2) + s * epb + t * eb
                    doff = soff
                else:
                    soff = chunk * e_pad + s * epb + t * eb
                    doff = s * epb + t * eb
                pltpu.sync_copy(srcs.at[pl.ds(soff, eb)], src_v.at[b])
                pltpu.sync_copy(dst.at[pl.ds(doff, eb)], dst_v.at[b])
                pltpu.async_copy(table.at[src_v.at[b]], rows_v.at[b], gsem)

            def drain_scatters(b):
                pltpu.make_async_copy(rows_v.at[b], acc.at[dst_v.at[b]],
                                      ssem).wait()

            load_and_gather(0, 0)

            def eblk(t, carry):
                b = lax.rem(t, 2)
                nb = lax.rem(t + 1, 2)

                @pl.when(t >= 1)
                def _():
                    drain_scatters(nb)

                @pl.when(t + 1 < nblk)
                def _():
                    load_and_gather(t + 1, nb)

                pltpu.make_async_copy(table.at[src_v.at[b]], rows_v.at[b],
                                      gsem).wait()
                pltpu.async_copy(rows_v.at[b], acc.at[dst_v.at[b]], ssem,
                                 add=True)
                return carry

            lax.fori_loop(0, nblk, eblk, 0)
            drain_scatters((nblk - 1) % 2)
            plsc.subcore_barrier()
            oc = c if split else chunk
            pltpu.sync_copy(acc.at[pl.ds(s * rps, rps)],
                            out.at[pl.ds(oc * n_pad + s * rps, rps)])
            if cc + 1 < cpc:
                plsc.subcore_barrier()

    return k


def _make_sc_degree(db):
    rps_l = LNP // 16
    rps_p = PNP // 16

    @functools.partial(
        pl.kernel,
        out_type=(jax.ShapeDtypeStruct((2 * LNP, 16), jnp.float32),
                  jax.ShapeDtypeStruct((2 * PNP, 16), jnp.float32)),
        mesh=plsc.VectorSubcoreMesh(**_MESH),
        compiler_params=pltpu.CompilerParams(use_tc_tiling_on_sc=False),
        scratch_types=[
            pltpu.VMEM((2, db), jnp.int32),
            pltpu.VMEM((db, 16), jnp.float32),
            pltpu.VMEM((_ZROWS, 16), jnp.float32),
            pltpu.VMEM_SHARED((PNP, 16), jnp.float32),
            pltpu.SemaphoreType.DMA,
        ],
    )
    def k(dst_l, dst_p, out_l, out_p, dst_v, ones_v, zbuf, acc, ssem):
        c = lax.axis_index("c")
        s = lax.axis_index("s")
        zero = jnp.zeros((16,), jnp.float32)
        one = jnp.ones((16,), jnp.float32)

        def fill(i, carry):
            zbuf[i] = zero
            return carry

        lax.fori_loop(0, _ZROWS, fill, 0)

        def fill1(i, carry):
            ones_v[i] = one
            return carry

        lax.fori_loop(0, db, fill1, 0)

        for dst, out, rps, e_pad in ((dst_l, out_l, rps_l, LEP),
                                     (dst_p, out_p, rps_p, PEP)):
            ept = e_pad // 32
            nblk = ept // db
            nz = rps // _ZROWS
            for z in range(nz):
                pltpu.sync_copy(zbuf,
                                acc.at[pl.ds(s * rps + z * _ZROWS, _ZROWS)])
            plsc.subcore_barrier()

            def load_idx(t, b):
                doff = (c * 16 + s) * ept + t * db
                pltpu.sync_copy(dst.at[pl.ds(doff, db)], dst_v.at[b])

            def drain(b):
                pltpu.make_async_copy(ones_v, acc.at[dst_v.at[b]],
                                      ssem).wait()

            load_idx(0, 0)

            def eblk(t, carry):
                b = lax.rem(t, 2)
                nb = lax.rem(t + 1, 2)

                @pl.when(t >= 1)
                def _():
                    drain(nb)

                @pl.when(t + 1 < nblk)
                def _():
                    load_idx(t + 1, nb)

                pltpu.async_copy(ones_v, acc.at[dst_v.at[b]], ssem, add=True)
                return carry

            lax.fori_loop(0, nblk, eblk, 0)
            drain((nblk - 1) % 2)
            plsc.subcore_barrier()
            n_pad = rps * 16
            pltpu.sync_copy(acc.at[pl.ds(s * rps, rps)],
                            out.at[pl.ds(c * n_pad + s * rps, rps)])
            plsc.subcore_barrier()

    return k


def _dinv_of(deg_ref):
    d = deg_ref[0, :, 0:1] + deg_ref[1, :, 0:1] + 1.0
    return lax.rsqrt(d)


def _tc_pre1(x_ref, w_ref, deg_ref, out_ref):
    dinv = _dinv_of(deg_ref)
    h = jnp.dot(x_ref[...], w_ref[...], preferred_element_type=jnp.float32)
    g = (h * dinv).astype(jnp.bfloat16)
    for cch in range(2):
        out_ref[cch] = g[:, cch * 32:(cch + 1) * 32]


def _tc_pre0(x_ref, deg_ref, out_ref):
    dinv = _dinv_of(deg_ref)
    out_ref[...] = (x_ref[...] * dinv).astype(jnp.bfloat16)


def _tc_mid(acc_ref, g_ref, deg_ref, w2_ref, b1_ref, out_ref):
    dinv = _dinv_of(deg_ref)
    a = jnp.concatenate(
        [(acc_ref[i] + g_ref[i]).astype(jnp.float32) for i in range(2)],
        axis=1)
    x1 = jax.nn.relu(dinv * a + b1_ref[...])
    g2 = jnp.dot(x1, w2_ref[...], preferred_element_type=jnp.float32) * dinv
    out_ref[...] = g2.astype(jnp.bfloat16)


def _tc_mid0(acc_ref, g_ref, deg_ref, w1_ref, w2_ref, b1_ref, out_ref):
    dinv = _dinv_of(deg_ref)
    a = (acc_ref[0] + acc_ref[1] + g_ref[...]).astype(jnp.float32)
    z = dinv * a
    x1 = jax.nn.relu(
        jnp.dot(z, w1_ref[...], preferred_element_type=jnp.float32)
        + b1_ref[...])
    g2 = jnp.dot(x1, w2_ref[...], preferred_element_type=jnp.float32) * dinv
    out_ref[...] = g2.astype(jnp.bfloat16)


def _make_tc_post(seg, rb):
    ngb = rb // seg

    def _tc_post(acc_ref, g_ref, deg_ref, b2_ref, out_ref):
        dinv = _dinv_of(deg_ref)
        a = (acc_ref[0] + acc_ref[1] + g_ref[...]).astype(jnp.float32)
        x2 = jax.nn.relu(dinv * a + b2_ref[...])
        rows = lax.broadcasted_iota(jnp.int32, (ngb, rb), 0)
        cols = lax.broadcasted_iota(jnp.int32, (ngb, rb), 1)
        pmat = jnp.where(cols // seg == rows, 1.0 / seg, 0.0)
        out_ref[...] = jnp.dot(pmat, x2, preferred_element_type=jnp.float32)

    return _tc_post


def _tc_head(xl_ref, xp_ref, w2d_ref, bilb_ref, aw1_ref, ab1_ref, aw2_ref,
             ab2_ref, ow1_ref, ob1_ref, ow2_ref, ob2_ref, fw1_ref, fb1_ref,
             fw2_ref, fb2_ref, fw3_ref, fb3_ref, out_ref):
    xl = xl_ref[...]
    xp = xp_ref[...]
    m = jnp.concatenate([xl[:, i:i + 1] * xp for i in range(32)], axis=1)
    bil = jax.nn.relu(
        jnp.dot(m, w2d_ref[...], preferred_element_type=jnp.float32)
        + bilb_ref[...])
    a1 = jax.nn.relu(
        jnp.dot(bil, aw1_ref[...], preferred_element_type=jnp.float32)
        + ab1_ref[...])
    att = jnp.sum(a1 * aw2_ref[...], axis=1, keepdims=True) + ab2_ref[...]
    attended = jax.nn.sigmoid(att) * bil
    f1 = jax.nn.relu(
        jnp.dot(attended, ow1_ref[...], preferred_element_type=jnp.float32)
        + ob1_ref[...])
    feat = jnp.dot(f1, ow2_ref[...],
                   preferred_element_type=jnp.float32) + ob2_ref[...]
    h = jax.nn.relu(
        jnp.dot(feat, fw1_ref[...], preferred_element_type=jnp.float32)
        + fb1_ref[...])
    h = jax.nn.relu(
        jnp.dot(h, fw2_ref[...], preferred_element_type=jnp.float32)
        + fb2_ref[...])
    out_ref[...] = jax.nn.sigmoid(
        jnp.sum(h * fw3_ref[...], axis=1, keepdims=True) + fb3_ref[...])


def _full(shape):
    return pl.BlockSpec(shape, lambda i: (0,) * len(shape))


def _prep_edges(ei, n, n_pad, e_pad):
    e = ei.shape[1]
    fill = jnp.full((e_pad - e,), jnp.int32(n))
    src = jnp.concatenate([ei[0], fill])
    dst2d = jnp.concatenate([ei[1], fill])

    def chunked(ch):
        return jnp.concatenate(
            [src + jnp.int32(cch * n_pad) for cch in range(ch)])

    return dst2d, chunked


def _pool(acc2, g2, deg, b2, n, n_pad, seg):
    rbp = seg * 8
    deg_spec_p = pl.BlockSpec((2, rbp, 16), lambda i: (0, i, 0))
    acc_spec = pl.BlockSpec((2, rbp, 32), lambda i: (0, i, 0))
    g_spec = pl.BlockSpec((rbp, 32), lambda i: (i, 0))
    return pl.pallas_call(
        _make_tc_post(seg, rbp),
        grid=(n // rbp,),
        in_specs=[acc_spec, g_spec, deg_spec_p, _full((1, 32))],
        out_specs=pl.BlockSpec((8, 32), lambda i: (i, 0)),
        out_shape=jax.ShapeDtypeStruct((NG, 32), jnp.float32),
    )(acc2, g2, deg, b2.reshape(1, 32))


def _ligand_branch(x, ei, W1, b1, W2, b2, deg):
    n, n_pad, e_pad, seg, rb = LN, LNP, LEP, LN // NG, 1000
    grid = n // rb
    dst2d, chunked = _prep_edges(ei, n, n_pad, e_pad)

    deg_spec = pl.BlockSpec((2, rb, 16), lambda i: (0, i, 0))
    t2w_spec = pl.BlockSpec((2, rb, 32), lambda i: (0, i, 0))
    t1w_spec = pl.BlockSpec((rb, 32), lambda i: (i, 0))

    g1 = pl.pallas_call(
        _tc_pre1,
        grid=(grid,),
        in_specs=[pl.BlockSpec((rb, 128), lambda i: (i, 0)),
                  _full((128, 64)), deg_spec],
        out_specs=t2w_spec,
        out_shape=jax.ShapeDtypeStruct((2, n_pad, 32), jnp.bfloat16),
    )(x, W1, deg)

    acc1 = _make_sc_scatter(n_pad, e_pad, 1024, 32, 1, False)(
        g1.reshape(2 * n_pad, 32), chunked(2), dst2d).reshape(2, n_pad, 32)

    g2 = pl.pallas_call(
        _tc_mid,
        grid=(grid,),
        in_specs=[t2w_spec, t2w_spec, deg_spec, _full((64, 32)),
                  _full((1, 64))],
        out_specs=t1w_spec,
        out_shape=jax.ShapeDtypeStruct((n_pad, 32), jnp.bfloat16),
    )(acc1, g1, deg, W2, b1.reshape(1, 64))

    acc2 = _make_sc_scatter(n_pad, e_pad, 1024, 32, 1, True)(
        g2, chunked(1), dst2d).reshape(2, n_pad, 32)

    return _pool(acc2, g2, deg, b2, n, n_pad, seg)


def _protein_branch(x, ei, W1, b1, W2, b2, deg):
    n, n_pad, e_pad, seg, rb = PN, PNP, PEP, PN // NG, 1000
    grid = n // rb
    dst2d, chunked = _prep_edges(ei, n, n_pad, e_pad)

    deg_spec = pl.BlockSpec((2, rb, 16), lambda i: (0, i, 0))
    t2w_spec = pl.BlockSpec((2, rb, 32), lambda i: (0, i, 0))
    t1w_spec = pl.BlockSpec((rb, 32), lambda i: (i, 0))

    g0 = pl.pallas_call(
        _tc_pre0,
        grid=(grid,),
        in_specs=[pl.BlockSpec((rb, 32), lambda i: (i, 0)), deg_spec],
        out_specs=t1w_spec,
        out_shape=jax.ShapeDtypeStruct((n_pad, 32), jnp.bfloat16),
    )(x, deg)

    acc0 = _make_sc_scatter(n_pad, e_pad, 256, 32, 1, True)(
        g0, chunked(1), dst2d).reshape(2, n_pad, 32)

    g2 = pl.pallas_call(
        _tc_mid0,
        grid=(grid,),
        in_specs=[t2w_spec, t1w_spec, deg_spec, _full((32, 64)),
                  _full((64, 32)), _full((1, 64))],
        out_specs=t1w_spec,
        out_shape=jax.ShapeDtypeStruct((n_pad, 32), jnp.bfloat16),
    )(acc0, g0, deg, W1, W2, b1.reshape(1, 64))

    acc2 = _make_sc_scatter(n_pad, e_pad, 256, 32, 1, True)(
        g2, chunked(1), dst2d).reshape(2, n_pad, 32)

    return _pool(acc2, g2, deg, b2, n, n_pad, seg)


def kernel(ligand_x, protein_x, ligand_edge_index, protein_edge_index,
           ligand_batch, protein_batch, lW1, lb1, lW2, lb2, pW1, pb1, pW2,
           pb2, bilW, bilb, attW1, attb1, attW2, attb2, outW1, outb1, outW2,
           outb2, finW1, finb1, finW2, finb2, finW3, finb3):
    del ligand_batch, protein_batch

    lx = jnp.pad(ligand_x, ((0, 0), (0, 128 - 78)))
    px = jnp.pad(protein_x, ((0, 0), (0, 32 - 30)))
    lW1p = jnp.pad(lW1, ((0, 128 - 78), (0, 0)))
    pW1p = jnp.pad(pW1, ((0, 32 - 30), (0, 0)))

    ldst = jnp.concatenate(
        [ligand_edge_index[1], jnp.full((LEP - LE,), jnp.int32(LN))])
    pdst = jnp.concatenate(
        [protein_edge_index[1], jnp.full((PEP - PE,), jnp.int32(PN))])
    deg_l, deg_p = _make_sc_degree(512)(ldst, pdst)
    deg_l = deg_l.reshape(2, LNP, 16)
    deg_p = deg_p.reshape(2, PNP, 16)

    xl = _ligand_branch(lx, ligand_edge_index, lW1p, lb1, lW2, lb2, deg_l)
    xp = _protein_branch(px, protein_edge_index, pW1p, pb1, pW2, pb2, deg_p)

    w2d = bilW.transpose(1, 2, 0).reshape(32 * 32, 128)
    rb = 200
    out = pl.pallas_call(
        _tc_head,
        grid=(NG // rb,),
        in_specs=[
            pl.BlockSpec((rb, 32), lambda i: (i, 0)),
            pl.BlockSpec((rb, 32), lambda i: (i, 0)),
            _full((1024, 128)),
            _full((1, 128)),
            _full((128, 64)),
            _full((1, 64)),
            _full((1, 64)),
            _full((1, 1)),
            _full((128, 128)),
            _full((1, 128)),
            _full((128, 64)),
            _full((1, 64)),
            _full((64, 128)),
            _full((1, 128)),
            _full((128, 64)),
            _full((1, 64)),
            _full((1, 64)),
            _full((1, 1)),
        ],
        out_specs=pl.BlockSpec((rb, 1), lambda i: (i, 0)),
        out_shape=jax.ShapeDtypeStruct((NG, 1), jnp.float32),
    )(xl, xp, w2d, bilb.reshape(1, 128), attW1, attb1.reshape(1, 64),
      attW2.reshape(1, 64), attb2.reshape(1, 1), outW1, outb1.reshape(1, 128),
      outW2, outb2.reshape(1, 64), finW1, finb1.reshape(1, 128), finW2,
      finb2.reshape(1, 64), finW3.reshape(1, 64), finb3.reshape(1, 1))
    return out


# speedup vs baseline: 1.3336x; 1.0096x over previous
import functools

import jax
import jax.numpy as jnp
from jax import lax
from jax.experimental import pallas as pl
from jax.experimental.pallas import tpu as pltpu
from jax.experimental.pallas import tpu_sc as plsc

NG = 1000
LN = 50000
PN = 100000
LE = 800000
PE = 1600000

LNP = 51200
PNP = 102400
LEP = 819200
PEP = 1638400

_MESH = dict(core_axis_name="c", subcore_axis_name="s", num_cores=2,
             num_subcores=16)

_ZROWS = 64


def _make_sc_scatter(n_pad, e_pad, eb, w, cpc, split, nbuf=2):
    nout = 2 if split else 2 * cpc
    epb = e_pad // 32 if split else e_pad // 16
    nblk = epb // eb
    rps = n_pad // 16
    nz = rps // _ZROWS

    @functools.partial(
        pl.kernel,
        out_type=jax.ShapeDtypeStruct((nout * n_pad, w), jnp.bfloat16),
        mesh=plsc.VectorSubcoreMesh(**_MESH),
        compiler_params=pltpu.CompilerParams(use_tc_tiling_on_sc=False),
        scratch_types=[
            pltpu.VMEM((nbuf, eb), jnp.int32),
            pltpu.VMEM((nbuf, eb), jnp.int32),
            pltpu.VMEM((nbuf, eb, w), jnp.bfloat16),
            pltpu.VMEM((_ZROWS, w), jnp.bfloat16),
            pltpu.VMEM_SHARED((n_pad, w), jnp.bfloat16),
            pltpu.SemaphoreType.DMA,
            pltpu.SemaphoreType.DMA,
        ],
    )
    def k(table, srcs, dst, out, src_v, dst_v, rows_v, zbuf, acc, gsem, ssem):
        c = lax.axis_index("c")
        s = lax.axis_index("s")
        zero = jnp.zeros((32,), jnp.bfloat16)

        def zb(i, carry):
            for q in range(w // 32):
                zbuf[i, pl.ds(q * 32, 32)] = zero
            return carry

        lax.fori_loop(0, _ZROWS, zb, 0)

        for cc in range(cpc):
            chunk = c * cpc + cc
            for z in range(nz):
                pltpu.sync_copy(zbuf, acc.at[pl.ds(s * rps + z * _ZROWS,
                                                   _ZROWS)])
            plsc.subcore_barrier()

            def load_and_gather(t, b):
                if split:
                    soff = c * (e_pad // 2) + s * epb + t * eb
                    doff = soff
                else:
                    soff = chunk * e_pad + s * epb + t * eb
                    doff = s * epb + t * eb
                pltpu.sync_copy(srcs.at[pl.ds(soff, eb)], src_v.at[b])
                pltpu.sync_copy(dst.at[pl.ds(doff, eb)], dst_v.at[b])
                pltpu.async_copy(table.at[src_v.at[b]], rows_v.at[b], gsem)

            def drain_scatters(b):
                pltpu.make_async_copy(rows_v.at[b], acc.at[dst_v.at[b]],
                                      ssem).wait()

            load_and_gather(0, 0)

            def eblk(t, carry):
                b = lax.rem(t, nbuf)
                nb = lax.rem(t + 1, nbuf)

                @pl.when(t >= nbuf - 1)
                def _():
                    drain_scatters(nb)

                @pl.when(t + 1 < nblk)
                def _():
                    load_and_gather(t + 1, nb)

                pltpu.make_async_copy(table.at[src_v.at[b]], rows_v.at[b],
                                      gsem).wait()
                pltpu.async_copy(rows_v.at[b], acc.at[dst_v.at[b]], ssem,
                                 add=True)
                return carry

            lax.fori_loop(0, nblk, eblk, 0)
            for q in range(max(0, nblk - (nbuf - 1)), nblk):
                drain_scatters(q % nbuf)
            plsc.subcore_barrier()
            oc = c if split else chunk
            pltpu.sync_copy(acc.at[pl.ds(s * rps, rps)],
                            out.at[pl.ds(oc * n_pad + s * rps, rps)])
            if cc + 1 < cpc:
                plsc.subcore_barrier()

    return k


def _make_sc_degree(db):
    rps_l = LNP // 16
    rps_p = PNP // 16

    @functools.partial(
        pl.kernel,
        out_type=(jax.ShapeDtypeStruct((2 * LNP, 16), jnp.float32),
                  jax.ShapeDtypeStruct((2 * PNP, 16), jnp.float32)),
        mesh=plsc.VectorSubcoreMesh(**_MESH),
        compiler_params=pltpu.CompilerParams(use_tc_tiling_on_sc=False),
        scratch_types=[
            pltpu.VMEM((2, db), jnp.int32),
            pltpu.VMEM((db, 16), jnp.float32),
            pltpu.VMEM((_ZROWS, 16), jnp.float32),
            pltpu.VMEM_SHARED((PNP, 16), jnp.float32),
            pltpu.SemaphoreType.DMA,
        ],
    )
    def k(dst_l, dst_p, out_l, out_p, dst_v, ones_v, zbuf, acc, ssem):
        c = lax.axis_index("c")
        s = lax.axis_index("s")
        zero = jnp.zeros((16,), jnp.float32)
        one = jnp.ones((16,), jnp.float32)

        def fill(i, carry):
            zbuf[i] = zero
            return carry

        lax.fori_loop(0, _ZROWS, fill, 0)

        def fill1(i, carry):
            ones_v[i] = one
            return carry

        lax.fori_loop(0, db, fill1, 0)

        for dst, out, rps, e_pad in ((dst_l, out_l, rps_l, LEP),
                                     (dst_p, out_p, rps_p, PEP)):
            ept = e_pad // 32
            nblk = ept // db
            nz = rps // _ZROWS
            for z in range(nz):
                pltpu.sync_copy(zbuf,
                                acc.at[pl.ds(s * rps + z * _ZROWS, _ZROWS)])
            plsc.subcore_barrier()

            def load_idx(t, b):
                doff = (c * 16 + s) * ept + t * db
                pltpu.sync_copy(dst.at[pl.ds(doff, db)], dst_v.at[b])

            def drain(b):
                pltpu.make_async_copy(ones_v, acc.at[dst_v.at[b]],
                                      ssem).wait()

            load_idx(0, 0)

            def eblk(t, carry):
                b = lax.rem(t, 2)
                nb = lax.rem(t + 1, 2)

                @pl.when(t >= 1)
                def _():
                    drain(nb)

                @pl.when(t + 1 < nblk)
                def _():
                    load_idx(t + 1, nb)

                pltpu.async_copy(ones_v, acc.at[dst_v.at[b]], ssem, add=True)
                return carry

            lax.fori_loop(0, nblk, eblk, 0)
            drain((nblk - 1) % 2)
            plsc.subcore_barrier()
            n_pad = rps * 16
            pltpu.sync_copy(acc.at[pl.ds(s * rps, rps)],
                            out.at[pl.ds(c * n_pad + s * rps, rps)])
            plsc.subcore_barrier()

    return k


def _dinv_of(deg_ref):
    d = deg_ref[0, :, 0:1] + deg_ref[1, :, 0:1] + 1.0
    return lax.rsqrt(d)


def _tc_pre1(x_ref, w_ref, deg_ref, out_ref):
    dinv = _dinv_of(deg_ref)
    h = jnp.dot(x_ref[...], w_ref[...], preferred_element_type=jnp.float32)
    g = (h * dinv).astype(jnp.bfloat16)
    for cch in range(2):
        out_ref[cch] = g[:, cch * 32:(cch + 1) * 32]


def _tc_pre0(x_ref, deg_ref, out_ref):
    dinv = _dinv_of(deg_ref)
    out_ref[...] = (x_ref[...] * dinv).astype(jnp.bfloat16)


def _tc_mid(acc_ref, g_ref, deg_ref, w2_ref, b1_ref, out_ref):
    dinv = _dinv_of(deg_ref)
    a = jnp.concatenate(
        [(acc_ref[i] + g_ref[i]).astype(jnp.float32) for i in range(2)],
        axis=1)
    x1 = jax.nn.relu(dinv * a + b1_ref[...])
    g2 = jnp.dot(x1, w2_ref[...], preferred_element_type=jnp.float32) * dinv
    out_ref[...] = g2.astype(jnp.bfloat16)


def _tc_mid0(acc_ref, g_ref, deg_ref, w1_ref, w2_ref, b1_ref, out_ref):
    dinv = _dinv_of(deg_ref)
    a = (acc_ref[0] + acc_ref[1] + g_ref[...]).astype(jnp.float32)
    z = dinv * a
    x1 = jax.nn.relu(
        jnp.dot(z, w1_ref[...], preferred_element_type=jnp.float32)
        + b1_ref[...])
    g2 = jnp.dot(x1, w2_ref[...], preferred_element_type=jnp.float32) * dinv
    out_ref[...] = g2.astype(jnp.bfloat16)


def _make_tc_post(seg, rb):
    ngb = rb // seg

    def _tc_post(acc_ref, g_ref, deg_ref, b2_ref, out_ref):
        dinv = _dinv_of(deg_ref)
        a = (acc_ref[0] + acc_ref[1] + g_ref[...]).astype(jnp.float32)
        x2 = jax.nn.relu(dinv * a + b2_ref[...])
        rows = lax.broadcasted_iota(jnp.int32, (ngb, rb), 0)
        cols = lax.broadcasted_iota(jnp.int32, (ngb, rb), 1)
        pmat = jnp.where(cols // seg == rows, 1.0 / seg, 0.0)
        out_ref[...] = jnp.dot(pmat, x2, preferred_element_type=jnp.float32)

    return _tc_post


def _tc_head(xl_ref, xp_ref, w2d_ref, bilb_ref, aw1_ref, ab1_ref, aw2_ref,
             ab2_ref, ow1_ref, ob1_ref, ow2_ref, ob2_ref, fw1_ref, fb1_ref,
             fw2_ref, fb2_ref, fw3_ref, fb3_ref, out_ref):
    xl = xl_ref[...]
    xp = xp_ref[...]
    m = jnp.concatenate([xl[:, i:i + 1] * xp for i in range(32)], axis=1)
    bil = jax.nn.relu(
        jnp.dot(m, w2d_ref[...], preferred_element_type=jnp.float32)
        + bilb_ref[...])
    a1 = jax.nn.relu(
        jnp.dot(bil, aw1_ref[...], preferred_element_type=jnp.float32)
        + ab1_ref[...])
    att = jnp.sum(a1 * aw2_ref[...], axis=1, keepdims=True) + ab2_ref[...]
    attended = jax.nn.sigmoid(att) * bil
    f1 = jax.nn.relu(
        jnp.dot(attended, ow1_ref[...], preferred_element_type=jnp.float32)
        + ob1_ref[...])
    feat = jnp.dot(f1, ow2_ref[...],
                   preferred_element_type=jnp.float32) + ob2_ref[...]
    h = jax.nn.relu(
        jnp.dot(feat, fw1_ref[...], preferred_element_type=jnp.float32)
        + fb1_ref[...])
    h = jax.nn.relu(
        jnp.dot(h, fw2_ref[...], preferred_element_type=jnp.float32)
        + fb2_ref[...])
    out_ref[...] = jax.nn.sigmoid(
        jnp.sum(h * fw3_ref[...], axis=1, keepdims=True) + fb3_ref[...])


def _full(shape):
    return pl.BlockSpec(shape, lambda i: (0,) * len(shape))


def _prep_edges(ei, n, n_pad, e_pad):
    e = ei.shape[1]
    fill = jnp.full((e_pad - e,), jnp.int32(n))
    src = jnp.concatenate([ei[0], fill])
    dst2d = jnp.concatenate([ei[1], fill])

    def chunked(ch):
        return jnp.concatenate(
            [src + jnp.int32(cch * n_pad) for cch in range(ch)])

    return dst2d, chunked


def _pool(acc2, g2, deg, b2, n, n_pad, seg):
    rbp = seg * 8
    deg_spec_p = pl.BlockSpec((2, rbp, 16), lambda i: (0, i, 0))
    acc_spec = pl.BlockSpec((2, rbp, 32), lambda i: (0, i, 0))
    g_spec = pl.BlockSpec((rbp, 32), lambda i: (i, 0))
    return pl.pallas_call(
        _make_tc_post(seg, rbp),
        grid=(n // rbp,),
        in_specs=[acc_spec, g_spec, deg_spec_p, _full((1, 32))],
        out_specs=pl.BlockSpec((8, 32), lambda i: (i, 0)),
        out_shape=jax.ShapeDtypeStruct((NG, 32), jnp.float32),
    )(acc2, g2, deg, b2.reshape(1, 32))


def _ligand_branch(x, ei, W1, b1, W2, b2, deg):
    n, n_pad, e_pad, seg, rb = LN, LNP, LEP, LN // NG, 1000
    grid = n // rb
    dst2d, chunked = _prep_edges(ei, n, n_pad, e_pad)

    deg_spec = pl.BlockSpec((2, rb, 16), lambda i: (0, i, 0))
    t2w_spec = pl.BlockSpec((2, rb, 32), lambda i: (0, i, 0))
    t1w_spec = pl.BlockSpec((rb, 32), lambda i: (i, 0))

    g1 = pl.pallas_call(
        _tc_pre1,
        grid=(grid,),
        in_specs=[pl.BlockSpec((rb, 128), lambda i: (i, 0)),
                  _full((128, 64)), deg_spec],
        out_specs=t2w_spec,
        out_shape=jax.ShapeDtypeStruct((2, n_pad, 32), jnp.bfloat16),
    )(x, W1, deg)

    acc1 = _make_sc_scatter(n_pad, e_pad, 1024, 32, 1, False, 3)(
        g1.reshape(2 * n_pad, 32), chunked(2), dst2d).reshape(2, n_pad, 32)

    g2 = pl.pallas_call(
        _tc_mid,
        grid=(grid,),
        in_specs=[t2w_spec, t2w_spec, deg_spec, _full((64, 32)),
                  _full((1, 64))],
        out_specs=t1w_spec,
        out_shape=jax.ShapeDtypeStruct((n_pad, 32), jnp.bfloat16),
    )(acc1, g1, deg, W2, b1.reshape(1, 64))

    acc2 = _make_sc_scatter(n_pad, e_pad, 1024, 32, 1, True, 3)(
        g2, chunked(1), dst2d).reshape(2, n_pad, 32)

    return _pool(acc2, g2, deg, b2, n, n_pad, seg)


def _protein_branch(x, ei, W1, b1, W2, b2, deg):
    n, n_pad, e_pad, seg, rb = PN, PNP, PEP, PN // NG, 1000
    grid = n // rb
    dst2d, chunked = _prep_edges(ei, n, n_pad, e_pad)

    deg_spec = pl.BlockSpec((2, rb, 16), lambda i: (0, i, 0))
    t2w_spec = pl.BlockSpec((2, rb, 32), lambda i: (0, i, 0))
    t1w_spec = pl.BlockSpec((rb, 32), lambda i: (i, 0))

    g0 = pl.pallas_call(
        _tc_pre0,
        grid=(grid,),
        in_specs=[pl.BlockSpec((rb, 32), lambda i: (i, 0)), deg_spec],
        out_specs=t1w_spec,
        out_shape=jax.ShapeDtypeStruct((n_pad, 32), jnp.bfloat16),
    )(x, deg)

    acc0 = _make_sc_scatter(n_pad, e_pad, 256, 32, 1, True, 3)(
        g0, chunked(1), dst2d).reshape(2, n_pad, 32)

    g2 = pl.pallas_call(
        _tc_mid0,
        grid=(grid,),
        in_specs=[t2w_spec, t1w_spec, deg_spec, _full((32, 64)),
                  _full((64, 32)), _full((1, 64))],
        out_specs=t1w_spec,
        out_shape=jax.ShapeDtypeStruct((n_pad, 32), jnp.bfloat16),
    )(acc0, g0, deg, W1, W2, b1.reshape(1, 64))

    acc2 = _make_sc_scatter(n_pad, e_pad, 256, 32, 1, True, 3)(
        g2, chunked(1), dst2d).reshape(2, n_pad, 32)

    return _pool(acc2, g2, deg, b2, n, n_pad, seg)


def kernel(ligand_x, protein_x, ligand_edge_index, protein_edge_index,
           ligand_batch, protein_batch, lW1, lb1, lW2, lb2, pW1, pb1, pW2,
           pb2, bilW, bilb, attW1, attb1, attW2, attb2, outW1, outb1, outW2,
           outb2, finW1, finb1, finW2, finb2, finW3, finb3):
    del ligand_batch, protein_batch

    lx = jnp.pad(ligand_x, ((0, 0), (0, 128 - 78)))
    px = jnp.pad(protein_x, ((0, 0), (0, 32 - 30)))
    lW1p = jnp.pad(lW1, ((0, 128 - 78), (0, 0)))
    pW1p = jnp.pad(pW1, ((0, 32 - 30), (0, 0)))

    ldst = jnp.concatenate(
        [ligand_edge_index[1], jnp.full((LEP - LE,), jnp.int32(LN))])
    pdst = jnp.concatenate(
        [protein_edge_index[1], jnp.full((PEP - PE,), jnp.int32(PN))])
    deg_l, deg_p = _make_sc_degree(512)(ldst, pdst)
    deg_l = deg_l.reshape(2, LNP, 16)
    deg_p = deg_p.reshape(2, PNP, 16)

    xl = _ligand_branch(lx, ligand_edge_index, lW1p, lb1, lW2, lb2, deg_l)
    xp = _protein_branch(px, protein_edge_index, pW1p, pb1, pW2, pb2, deg_p)

    w2d = bilW.transpose(1, 2, 0).reshape(32 * 32, 128)
    rb = 200
    out = pl.pallas_call(
        _tc_head,
        grid=(NG // rb,),
        in_specs=[
            pl.BlockSpec((rb, 32), lambda i: (i, 0)),
            pl.BlockSpec((rb, 32), lambda i: (i, 0)),
            _full((1024, 128)),
            _full((1, 128)),
            _full((128, 64)),
            _full((1, 64)),
            _full((1, 64)),
            _full((1, 1)),
            _full((128, 128)),
            _full((1, 128)),
            _full((128, 64)),
            _full((1, 64)),
            _full((64, 128)),
            _full((1, 128)),
            _full((128, 64)),
            _full((1, 64)),
            _full((1, 64)),
            _full((1, 1)),
        ],
        out_specs=pl.BlockSpec((rb, 1), lambda i: (i, 0)),
        out_shape=jax.ShapeDtypeStruct((NG, 1), jnp.float32),
    )(xl, xp, w2d, bilb.reshape(1, 128), attW1, attb1.reshape(1, 64),
      attW2.reshape(1, 64), attb2.reshape(1, 1), outW1, outb1.reshape(1, 128),
      outW2, outb2.reshape(1, 64), finW1, finb1.reshape(1, 128), finW2,
      finb2.reshape(1, 64), finW3.reshape(1, 64), finb3.reshape(1, 1))
    return out
